# baseline (speedup 1.0000x reference)
import functools

import jax
import jax.numpy as jnp
from jax import lax
from jax.experimental import pallas as pl
from jax.experimental.pallas import tpu as pltpu
from jax.experimental.pallas import tpu_sc as plsc

_NR = 16


def _build_lookup(B, V, D):
    info = plsc.get_sparse_core_info()
    nw = info.num_cores * info.num_subcores
    assert B % nw == 0
    b_per_w = B // nw

    mesh = plsc.VectorSubcoreMesh(core_axis_name="c", subcore_axis_name="s")

    cp = pltpu.CompilerParams()
    if "needs_layout_passes" in pltpu.CompilerParams.__dataclass_fields__:
        import dataclasses

        cp = dataclasses.replace(cp, needs_layout_passes=False)

    @functools.partial(
        pl.kernel,
        mesh=mesh,
        compiler_params=cp,
        out_type=jax.ShapeDtypeStruct((D, B), jnp.float32),
        scratch_types=(
            [pltpu.VMEM((b_per_w,), jnp.int32)]
            + [pltpu.VMEM((D, b_per_w), jnp.float32)]
            + [pltpu.VMEM((D, 128), jnp.float32) for _ in range(_NR)]
            + [pltpu.SemaphoreType.DMA for _ in range(_NR)]
        ),
    )
    def lookup(tablet_hbm, idx_hbm, outt_hbm, idx_v, feat_v, *slots_and_sems):
        slots = slots_and_sems[:_NR]
        sems = slots_and_sems[_NR:]
        wid = lax.axis_index("s") * info.num_cores + lax.axis_index("c")
        base = wid * b_per_w
        pltpu.sync_copy(idx_hbm.at[pl.ds(base, b_per_w)], idx_v)

        row_lo = lax.iota(jnp.int32, 16)
        row_hi = row_lo + 16

        def fetch(r, slot, sem):
            q = pl.multiple_of((r // 128) * 128, 128)
            return pltpu.async_copy(
                tablet_hbm.at[:, pl.ds(q, 128)], slot, sem
            )

        def extract(r, k, slot):
            lane = jnp.full((16,), r % 128, jnp.int32)
            col = jnp.full((16,), k, jnp.int32)
            lo = plsc.load_gather(slot, [row_lo, lane])
            hi = plsc.load_gather(slot, [row_hi, lane])
            plsc.store_scatter(feat_v, [row_lo, col], lo)
            plsc.store_scatter(feat_v, [row_hi, col], hi)

        def body(j, _):
            vec = idx_v[pl.ds(j * _NR, _NR)]
            copies = []
            for k in range(_NR):
                copies.append(fetch(vec[k], slots[k], sems[k]))
            for k in range(_NR):
                copies[k].wait()
                extract(vec[k], j * _NR + k, slots[k])
            return 0

        lax.fori_loop(0, b_per_w // _NR, body, 0)
        pltpu.sync_copy(feat_v, outt_hbm.at[:, pl.ds(base, b_per_w)])

    return lookup


def kernel(label, table):
    flat = label.reshape(-1).astype(jnp.int32)
    V, D = table.shape
    outt = _build_lookup(flat.shape[0], V, D)(table.T, flat)
    return outt.T[..., None]
